# baseline (speedup 1.0000x reference)
import functools

import jax
import jax.numpy as jnp
from jax import lax
from jax.experimental import pallas as pl
from jax.experimental.pallas import tpu as pltpu
from jax.experimental.pallas import tpu_sc as plsc

K = 8192
D = 32
N = 8192
RB = 128
KB = 2048
LT = 128
NLT = KB // LT
NSTEPS = N // RB
NKC = K // KB

NC = 2
NS = 16
NW = NC * NS
BPW = N // NW


def _bf16_round(x):
    b = lax.bitcast_convert_type(x, jnp.uint32)
    b = (b + 0x7FFF + ((b >> 16) & 1)) & jnp.uint32(0xFFFF0000)
    return lax.bitcast_convert_type(b, jnp.float32)


def _tc_body(z_ref, e_ref, m_ref, idx_ref, loss_ref, acc_ref):
    step = pl.program_id(0)

    @pl.when(step == 0)
    def _init():
        acc_ref[0] = 0.0
        acc_ref[1] = 0.0

    z = z_ref[...]
    zsq = jnp.sum(z * z, axis=1)
    zm2 = _bf16_round(z * (-2.0)).astype(jnp.bfloat16)

    lane = lax.broadcasted_iota(jnp.int32, (RB, LT), 1)

    def chunk(j, carry):
        acc_cmp, acc_exact, acc_idx = carry
        e = e_ref[pl.ds(j * KB, KB), :]
        esq = jnp.sum(e * e, axis=1)
        m2 = lax.dot_general(zm2, e, (((1,), (1,)), ((), ())),
                             preferred_element_type=jnp.float32)
        vmin = jnp.full((RB, LT), jnp.inf, dtype=jnp.float32)
        vidx = jnp.zeros((RB, LT), dtype=jnp.int32)
        for t in range(NLT):
            sl = slice(t * LT, (t + 1) * LT)
            d_t = (zsq[:, None] + esq[None, sl]) + m2[:, sl]
            lt_m = d_t < vmin
            vmin = jnp.where(lt_m, d_t, vmin)
            vidx = jnp.where(lt_m, lane + (t * LT), vidx)
        v = jnp.min(vmin, axis=1)
        i = jnp.min(jnp.where(vmin == v[:, None], vidx, K), axis=1) + j * KB
        keep = (acc_cmp < v) | ((acc_cmp == v) & (acc_idx < i))
        acc_idx = jnp.where(keep, acc_idx, i)
        acc_exact = jnp.where(keep, acc_exact, v)
        acc_cmp = _bf16_round(jnp.where(keep, acc_cmp, v))
        return (acc_cmp, acc_exact, acc_idx)

    inf = jnp.full((RB,), jnp.inf, dtype=jnp.float32)
    zero = jnp.zeros((RB,), dtype=jnp.int32)
    _, runmin, runidx = lax.fori_loop(0, NKC, chunk, (inf, inf, zero))

    idx_ref[0, 0, :] = runidx
    msk = m_ref[0, 0, :]
    acc_ref[0] += jnp.sum(msk * runmin)
    acc_ref[1] += jnp.sum(msk)

    @pl.when(step == NSTEPS - 1)
    def _finalize():
        se_sum = acc_ref[0] / jnp.float32(D)
        loss_ref[0, 0] = 1.25 * (se_sum / jnp.maximum(acc_ref[1], 1.0))


@functools.cache
def _make_tc_call():
    return pl.pallas_call(
        _tc_body,
        grid=(NSTEPS,),
        in_specs=[
            pl.BlockSpec((RB, D), lambda i: (i, 0)),
            pl.BlockSpec((K, D), lambda i: (0, 0)),
            pl.BlockSpec((1, 1, RB), lambda i: (i, 0, 0)),
        ],
        out_specs=[
            pl.BlockSpec((1, 1, RB), lambda i: (i, 0, 0)),
            pl.BlockSpec(memory_space=pltpu.SMEM),
        ],
        out_shape=[
            jax.ShapeDtypeStruct((NSTEPS, 1, RB), jnp.int32),
            jax.ShapeDtypeStruct((1, 1), jnp.float32),
        ],
        scratch_shapes=[pltpu.SMEM((2,), jnp.float32)],
    )


def _sc_body(table_hbm, idx_hbm, z_hbm, out_hbm, idx_v, q_v, z_v, sem):
    wid = lax.axis_index("s") * NC + lax.axis_index("c")
    base = wid * BPW
    pltpu.sync_copy(idx_hbm.at[pl.ds(base, BPW)], idx_v)
    gather = pltpu.async_copy(table_hbm.at[idx_v], q_v, sem)
    pltpu.sync_copy(z_hbm.at[pl.ds(base, BPW)], z_v)
    gather.wait()

    def row(i, _):
        for h in range(D // 16):
            sl = pl.ds(h * 16, 16)
            q = q_v[i, sl]
            zz = z_v[i, sl]
            q_v[i, sl] = zz + (q - zz)
        return 0

    lax.fori_loop(0, BPW, row, 0)
    pltpu.sync_copy(q_v, out_hbm.at[pl.ds(base, BPW)])


@functools.cache
def _make_sc_call():
    return pl.kernel(
        _sc_body,
        mesh=plsc.VectorSubcoreMesh(core_axis_name="c", subcore_axis_name="s"),
        out_type=jax.ShapeDtypeStruct((N, D), jnp.float32),
        scratch_types=[
            pltpu.VMEM((BPW,), jnp.int32),
            pltpu.VMEM((BPW, D), jnp.float32),
            pltpu.VMEM((BPW, D), jnp.float32),
            pltpu.SemaphoreType.DMA,
        ],
        compiler_params=pltpu.CompilerParams(use_tc_tiling_on_sc=False),
    )


def kernel(z, mask, embeddings):
    orig_shape = z.shape
    z_flat = z.reshape(N, D)
    mask3 = mask.reshape(NSTEPS, 1, RB)
    idx3, loss11 = _make_tc_call()(z_flat, embeddings, mask3)
    idx_flat = idx3.reshape(N)
    out_flat = _make_sc_call()(embeddings, idx_flat, z_flat)
    quantized_st = out_flat.reshape(orig_shape)
    quantized_indices = idx_flat.reshape(orig_shape[:-1])
    loss = loss11[0, 0]
    return quantized_st, quantized_indices, loss


# speedup vs baseline: 1.4252x; 1.4252x over previous
import functools

import jax
import jax.numpy as jnp
from jax import lax
from jax.experimental import pallas as pl
from jax.experimental.pallas import tpu as pltpu
from jax.experimental.pallas import tpu_sc as plsc

K = 8192
D = 32
N = 8192
RB = 256
KB = 2048
LT = 128
NLT = KB // LT
NSTEPS = N // RB
NKC = K // KB

NC = 2
NS = 16
NW = NC * NS
BPW = N // NW


def _bf16_round(x):
    b = lax.bitcast_convert_type(x, jnp.uint32)
    b = (b + 0x7FFF + ((b >> 16) & 1)) & jnp.uint32(0xFFFF0000)
    return lax.bitcast_convert_type(b, jnp.float32)


def _tc_body(z_ref, e_ref, m_ref, idx_ref, loss_ref, acc_ref):
    step = pl.program_id(0)

    @pl.when(step == 0)
    def _init():
        acc_ref[0] = 0.0
        acc_ref[1] = 0.0

    z = z_ref[...]
    zsq = jnp.sum(z * z, axis=1)
    zm2 = _bf16_round(z * (-2.0)).astype(jnp.bfloat16)

    lane = lax.broadcasted_iota(jnp.int32, (RB, LT), 1)

    def chunk(j, carry):
        acc_cmp, acc_exact, acc_idx = carry
        e = e_ref[pl.ds(j * KB, KB), :]
        esq = jnp.sum(e * e, axis=1)
        m2 = lax.dot_general(zm2, e, (((1,), (1,)), ((), ())),
                             preferred_element_type=jnp.float32)
        groups = []
        for g in range(4):
            gmin = gidx = None
            for t in range(4 * g, 4 * g + 4):
                sl = slice(t * LT, (t + 1) * LT)
                d_t = (zsq[:, None] + esq[None, sl]) + m2[:, sl]
                if gmin is None:
                    gmin, gidx = d_t, lane + (t * LT)
                else:
                    lt_m = d_t < gmin
                    gmin = jnp.where(lt_m, d_t, gmin)
                    gidx = jnp.where(lt_m, lane + (t * LT), gidx)
            groups.append((gmin, gidx))
        vmin, vidx = groups[0]
        for gmin, gidx in groups[1:]:
            lt_m = gmin < vmin
            vmin = jnp.where(lt_m, gmin, vmin)
            vidx = jnp.where(lt_m, gidx, vidx)
        v = jnp.min(vmin, axis=1)
        i = jnp.min(jnp.where(vmin == v[:, None], vidx, K), axis=1) + j * KB
        keep = (acc_cmp < v) | ((acc_cmp == v) & (acc_idx < i))
        acc_idx = jnp.where(keep, acc_idx, i)
        acc_exact = jnp.where(keep, acc_exact, v)
        acc_cmp = _bf16_round(jnp.where(keep, acc_cmp, v))
        return (acc_cmp, acc_exact, acc_idx)

    inf = jnp.full((RB,), jnp.inf, dtype=jnp.float32)
    zero = jnp.zeros((RB,), dtype=jnp.int32)
    _, runmin, runidx = lax.fori_loop(0, NKC, chunk, (inf, inf, zero))

    idx_ref[0, 0, :] = runidx
    msk = m_ref[0, 0, :]
    acc_ref[0] += jnp.sum(msk * runmin)
    acc_ref[1] += jnp.sum(msk)

    @pl.when(step == NSTEPS - 1)
    def _finalize():
        se_sum = acc_ref[0] / jnp.float32(D)
        loss_ref[0, 0] = 1.25 * (se_sum / jnp.maximum(acc_ref[1], 1.0))


@functools.cache
def _make_tc_call():
    return pl.pallas_call(
        _tc_body,
        grid=(NSTEPS,),
        in_specs=[
            pl.BlockSpec((RB, D), lambda i: (i, 0)),
            pl.BlockSpec((K, D), lambda i: (0, 0)),
            pl.BlockSpec((1, 1, RB), lambda i: (i, 0, 0)),
        ],
        out_specs=[
            pl.BlockSpec((1, 1, RB), lambda i: (i, 0, 0)),
            pl.BlockSpec(memory_space=pltpu.SMEM),
        ],
        out_shape=[
            jax.ShapeDtypeStruct((NSTEPS, 1, RB), jnp.int32),
            jax.ShapeDtypeStruct((1, 1), jnp.float32),
        ],
        scratch_shapes=[pltpu.SMEM((2,), jnp.float32)],
    )


def _sc_body(table_hbm, idx_hbm, z_hbm, out_hbm, idx_v, q_v, z_v, sem):
    wid = lax.axis_index("s") * NC + lax.axis_index("c")
    base = wid * BPW
    pltpu.sync_copy(idx_hbm.at[pl.ds(base, BPW)], idx_v)
    gather = pltpu.async_copy(table_hbm.at[idx_v], q_v, sem)
    pltpu.sync_copy(z_hbm.at[pl.ds(base, BPW)], z_v)
    gather.wait()

    def row(i, _):
        for h in range(D // 16):
            sl = pl.ds(h * 16, 16)
            q = q_v[i, sl]
            zz = z_v[i, sl]
            q_v[i, sl] = zz + (q - zz)
        return 0

    lax.fori_loop(0, BPW, row, 0)
    pltpu.sync_copy(q_v, out_hbm.at[pl.ds(base, BPW)])


@functools.cache
def _make_sc_call():
    return pl.kernel(
        _sc_body,
        mesh=plsc.VectorSubcoreMesh(core_axis_name="c", subcore_axis_name="s"),
        out_type=jax.ShapeDtypeStruct((N, D), jnp.float32),
        scratch_types=[
            pltpu.VMEM((BPW,), jnp.int32),
            pltpu.VMEM((BPW, D), jnp.float32),
            pltpu.VMEM((BPW, D), jnp.float32),
            pltpu.SemaphoreType.DMA,
        ],
        compiler_params=pltpu.CompilerParams(use_tc_tiling_on_sc=False),
    )


def kernel(z, mask, embeddings):
    orig_shape = z.shape
    z_flat = z.reshape(N, D)
    mask3 = mask.reshape(NSTEPS, 1, RB)
    idx3, loss11 = _make_tc_call()(z_flat, embeddings, mask3)
    idx_flat = idx3.reshape(N)
    out_flat = _make_sc_call()(embeddings, idx_flat, z_flat)
    quantized_st = out_flat.reshape(orig_shape)
    quantized_indices = idx_flat.reshape(orig_shape[:-1])
    loss = loss11[0, 0]
    return quantized_st, quantized_indices, loss


# speedup vs baseline: 1.5541x; 1.0905x over previous
import functools

import jax
import jax.numpy as jnp
from jax import lax
from jax.experimental import pallas as pl
from jax.experimental.pallas import tpu as pltpu
from jax.experimental.pallas import tpu_sc as plsc

K = 8192
D = 32
N = 8192
RB = 256
KB = 2048
LT = 128
NLT = KB // LT
NSTEPS = N // RB
NKC = K // KB

NC = 2
NS = 16
NW = NC * NS
BPW = N // NW


def _bf16_round(x):
    b = lax.bitcast_convert_type(x, jnp.uint32)
    b = (b + 0x7FFF + ((b >> 16) & 1)) & jnp.uint32(0xFFFF0000)
    return lax.bitcast_convert_type(b, jnp.float32)


def _tc_body(z_ref, e_ref, m_ref, idx_ref, loss_ref, esq_ref, acc_ref):
    step = pl.program_id(0)

    @pl.when(step == 0)
    def _init():
        e_all = e_ref[...]
        esq_ref[...] = jnp.sum(e_all * e_all, axis=1)
        acc_ref[...] = jnp.zeros((2, RB), dtype=jnp.float32)

    z = z_ref[...]
    zsq = jnp.sum(z * z, axis=1)
    zm2 = _bf16_round(z * (-2.0)).astype(jnp.bfloat16)

    lane = lax.broadcasted_iota(jnp.int32, (RB, LT), 1)

    def chunk(j, carry):
        acc_cmp, acc_exact, acc_idx = carry
        e = e_ref[pl.ds(j * KB, KB), :]
        esq = esq_ref[pl.ds(j * KB, KB)]
        m2 = lax.dot_general(zm2, e, (((1,), (1,)), ((), ())),
                             preferred_element_type=jnp.float32)
        groups = []
        for g in range(4):
            gmin = gidx = None
            for t in range(4 * g, 4 * g + 4):
                sl = slice(t * LT, (t + 1) * LT)
                d_t = (zsq[:, None] + esq[None, sl]) + m2[:, sl]
                if gmin is None:
                    gmin, gidx = d_t, lane + (t * LT)
                else:
                    lt_m = d_t < gmin
                    gmin = jnp.where(lt_m, d_t, gmin)
                    gidx = jnp.where(lt_m, lane + (t * LT), gidx)
            groups.append((gmin, gidx))
        vmin, vidx = groups[0]
        for gmin, gidx in groups[1:]:
            lt_m = gmin < vmin
            vmin = jnp.where(lt_m, gmin, vmin)
            vidx = jnp.where(lt_m, gidx, vidx)
        v = jnp.min(vmin, axis=1)
        i = jnp.min(jnp.where(vmin == v[:, None], vidx, K), axis=1) + j * KB
        keep = (acc_cmp < v) | ((acc_cmp == v) & (acc_idx < i))
        acc_idx = jnp.where(keep, acc_idx, i)
        acc_exact = jnp.where(keep, acc_exact, v)
        acc_cmp = _bf16_round(jnp.where(keep, acc_cmp, v))
        return (acc_cmp, acc_exact, acc_idx)

    inf = jnp.full((RB,), jnp.inf, dtype=jnp.float32)
    zero = jnp.zeros((RB,), dtype=jnp.int32)
    _, runmin, runidx = lax.fori_loop(0, NKC, chunk, (inf, inf, zero))

    idx_ref[0, 0, :] = runidx
    msk = m_ref[0, 0, :]
    acc_ref[0, :] += msk * runmin
    acc_ref[1, :] += msk

    @pl.when(step == NSTEPS - 1)
    def _finalize():
        se_sum = jnp.sum(acc_ref[0, :]) / jnp.float32(D)
        msum = jnp.sum(acc_ref[1, :])
        loss_ref[0, 0] = 1.25 * (se_sum / jnp.maximum(msum, 1.0))


@functools.cache
def _make_tc_call():
    return pl.pallas_call(
        _tc_body,
        grid=(NSTEPS,),
        in_specs=[
            pl.BlockSpec((RB, D), lambda i: (i, 0)),
            pl.BlockSpec((K, D), lambda i: (0, 0)),
            pl.BlockSpec((1, 1, RB), lambda i: (i, 0, 0)),
        ],
        out_specs=[
            pl.BlockSpec((1, 1, RB), lambda i: (i, 0, 0)),
            pl.BlockSpec(memory_space=pltpu.SMEM),
        ],
        out_shape=[
            jax.ShapeDtypeStruct((NSTEPS, 1, RB), jnp.int32),
            jax.ShapeDtypeStruct((1, 1), jnp.float32),
        ],
        scratch_shapes=[
            pltpu.VMEM((K,), jnp.float32),
            pltpu.VMEM((2, RB), jnp.float32),
        ],
    )


def _sc_body(table_hbm, idx_hbm, z_hbm, out_hbm, idx_v, q_v, z_v, sem):
    wid = lax.axis_index("s") * NC + lax.axis_index("c")
    base = wid * BPW
    pltpu.sync_copy(idx_hbm.at[pl.ds(base, BPW)], idx_v)
    gather = pltpu.async_copy(table_hbm.at[idx_v], q_v, sem)
    pltpu.sync_copy(z_hbm.at[pl.ds(base, BPW)], z_v)
    gather.wait()

    def row(i, _):
        for h in range(D // 16):
            sl = pl.ds(h * 16, 16)
            q = q_v[i, sl]
            zz = z_v[i, sl]
            q_v[i, sl] = zz + (q - zz)
        return 0

    lax.fori_loop(0, BPW, row, 0)
    pltpu.sync_copy(q_v, out_hbm.at[pl.ds(base, BPW)])


@functools.cache
def _make_sc_call():
    return pl.kernel(
        _sc_body,
        mesh=plsc.VectorSubcoreMesh(core_axis_name="c", subcore_axis_name="s"),
        out_type=jax.ShapeDtypeStruct((N, D), jnp.float32),
        scratch_types=[
            pltpu.VMEM((BPW,), jnp.int32),
            pltpu.VMEM((BPW, D), jnp.float32),
            pltpu.VMEM((BPW, D), jnp.float32),
            pltpu.SemaphoreType.DMA,
        ],
        compiler_params=pltpu.CompilerParams(use_tc_tiling_on_sc=False),
    )


def kernel(z, mask, embeddings):
    orig_shape = z.shape
    z_flat = z.reshape(N, D)
    mask3 = mask.reshape(NSTEPS, 1, RB)
    idx3, loss11 = _make_tc_call()(z_flat, embeddings, mask3)
    idx_flat = idx3.reshape(N)
    out_flat = _make_sc_call()(embeddings, idx_flat, z_flat)
    quantized_st = out_flat.reshape(orig_shape)
    quantized_indices = idx_flat.reshape(orig_shape[:-1])
    loss = loss11[0, 0]
    return quantized_st, quantized_indices, loss
